# baseline (speedup 1.0000x reference)
import functools

import jax
import jax.numpy as jnp
from jax import lax
from jax.experimental import pallas as pl
from jax.experimental.pallas import tpu as pltpu
from jax.experimental.pallas import tpu_sc as plsc

B = 16384
D = 64
HID = 64
NC = 2
NS = 16
NW = NC * NS
BPW = B // NW

TB = 2048


def _sc_gather(table, ids):
    mesh = plsc.VectorSubcoreMesh(core_axis_name="c", subcore_axis_name="s")

    @functools.partial(
        pl.kernel,
        mesh=mesh,
        out_type=jax.ShapeDtypeStruct((B, D), jnp.float32),
        compiler_params=pltpu.CompilerParams(use_tc_tiling_on_sc=False),
        scratch_types=[
            pltpu.VMEM((BPW,), jnp.int32),
            pltpu.VMEM((BPW, D), jnp.float32),
            pltpu.SemaphoreType.DMA,
        ],
    )
    def gather_kernel(table_hbm, idx_hbm, out_hbm, idx_v, rows_v, sem):
        wid = lax.axis_index("s") * NC + lax.axis_index("c")
        base = wid * BPW
        pltpu.sync_copy(idx_hbm.at[pl.ds(base, BPW)], idx_v)
        pltpu.async_copy(table_hbm.at[idx_v], rows_v, sem).wait()
        pltpu.sync_copy(rows_v, out_hbm.at[pl.ds(base, BPW)])

    return gather_kernel(table, ids)


def _combine_body(ct_ref, m_ref, g_ref, w1t_ref, b1_ref, w2t_ref, b2_ref,
                  out_ref):
    h = jnp.dot(w1t_ref[...], ct_ref[...],
                preferred_element_type=jnp.float32) + b1_ref[...]
    h = jnp.maximum(h, 0.0)
    spa = jnp.dot(w2t_ref[...], h,
                  preferred_element_type=jnp.float32) + b2_ref[...]
    gt = g_ref[...].T
    v = jnp.where(m_ref[...] > 0.0, gt, spa)
    inv = lax.rsqrt(jnp.sum(v * v, axis=0, keepdims=True))
    out_ref[...] = v * inv


def _tc_combine(ct, m, g, w1t, b1, w2t, b2):
    return pl.pallas_call(
        _combine_body,
        grid=(B // TB,),
        in_specs=[
            pl.BlockSpec((2, TB), lambda i: (0, i)),
            pl.BlockSpec((1, TB), lambda i: (0, i)),
            pl.BlockSpec((TB, D), lambda i: (i, 0)),
            pl.BlockSpec((HID, 2), lambda i: (0, 0)),
            pl.BlockSpec((HID, 1), lambda i: (0, 0)),
            pl.BlockSpec((D, HID), lambda i: (0, 0)),
            pl.BlockSpec((D, 1), lambda i: (0, 0)),
        ],
        out_specs=pl.BlockSpec((D, TB), lambda i: (0, i)),
        out_shape=jax.ShapeDtypeStruct((D, B), jnp.float32),
    )(ct, m, g, w1t, b1, w2t, b2)


def kernel(coords, nogeo_khot, spa_W1, spa_b1, spa_W2, spa_b2, nogeo_table,
           nogeo_ids):
    g = _sc_gather(nogeo_table, nogeo_ids)
    ct = coords.reshape(B, 2).T
    m = nogeo_khot.reshape(1, B)
    return _tc_combine(ct, m, g, spa_W1.T, spa_b1.reshape(HID, 1),
                       spa_W2.T, spa_b2.reshape(D, 1))


# speedup vs baseline: 1.6874x; 1.6874x over previous
import functools

import jax
import jax.numpy as jnp
from jax import lax
from jax.experimental import pallas as pl
from jax.experimental.pallas import tpu as pltpu
from jax.experimental.pallas import tpu_sc as plsc

B = 16384
D = 64
HID = 64
NC = 2
NS = 16
NW = NC * NS
BPW = B // NW

TB = 2048


def _sc_gather(table, ids):
    K = 32
    mesh = plsc.VectorSubcoreMesh(core_axis_name="c", subcore_axis_name="s")

    @functools.partial(
        pl.kernel,
        mesh=mesh,
        out_type=jax.ShapeDtypeStruct((B, D), jnp.float32),
        scratch_types=[
            pltpu.VMEM((BPW,), jnp.int32),
            pltpu.VMEM((BPW, D), jnp.float32),
            pltpu.SemaphoreType.DMA,
        ],
    )
    def gather_kernel(table_hbm, idx_hbm, out_hbm, idx_v, rows_v, sem):
        wid = lax.axis_index("s") * NC + lax.axis_index("c")
        base = wid * BPW
        pltpu.sync_copy(idx_hbm.at[pl.ds(base, BPW)], idx_v)

        def chunk(c, carry):
            b0 = c * K
            copies = []
            for g in range(K // 16):
                vec = idx_v[pl.ds(b0 + g * 16, 16)]
                for k in range(16):
                    copies.append(pltpu.make_async_copy(
                        table_hbm.at[pl.ds(vec[k], 1), :],
                        rows_v.at[pl.ds(b0 + g * 16 + k, 1), :],
                        sem,
                    ))
            for cp in copies:
                cp.start()
            for cp in copies:
                cp.wait()
            return carry

        lax.fori_loop(0, BPW // K, chunk, 0)
        pltpu.sync_copy(rows_v, out_hbm.at[pl.ds(base, BPW)])

    return gather_kernel(table, ids)


def _combine_body(ct_ref, m_ref, g_ref, w1t_ref, b1_ref, w2t_ref, b2_ref,
                  out_ref):
    h = jnp.dot(w1t_ref[...], ct_ref[...],
                preferred_element_type=jnp.float32) + b1_ref[...]
    h = jnp.maximum(h, 0.0)
    spa = jnp.dot(w2t_ref[...], h,
                  preferred_element_type=jnp.float32) + b2_ref[...]
    gt = g_ref[...].T
    v = jnp.where(m_ref[...] > 0.0, gt, spa)
    inv = lax.rsqrt(jnp.sum(v * v, axis=0, keepdims=True))
    out_ref[...] = v * inv


def _tc_combine(ct, m, g, w1t, b1, w2t, b2):
    return pl.pallas_call(
        _combine_body,
        grid=(B // TB,),
        in_specs=[
            pl.BlockSpec((2, TB), lambda i: (0, i)),
            pl.BlockSpec((1, TB), lambda i: (0, i)),
            pl.BlockSpec((TB, D), lambda i: (i, 0)),
            pl.BlockSpec((HID, 2), lambda i: (0, 0)),
            pl.BlockSpec((HID, 1), lambda i: (0, 0)),
            pl.BlockSpec((D, HID), lambda i: (0, 0)),
            pl.BlockSpec((D, 1), lambda i: (0, 0)),
        ],
        out_specs=pl.BlockSpec((D, TB), lambda i: (0, i)),
        out_shape=jax.ShapeDtypeStruct((D, B), jnp.float32),
    )(ct, m, g, w1t, b1, w2t, b2)


def kernel(coords, nogeo_khot, spa_W1, spa_b1, spa_W2, spa_b2, nogeo_table,
           nogeo_ids):
    g = _sc_gather(nogeo_table, nogeo_ids)
    ct = coords.reshape(B, 2).T
    m = nogeo_khot.reshape(1, B)
    return _tc_combine(ct, m, g, spa_W1.T, spa_b1.reshape(HID, 1),
                       spa_W2.T, spa_b2.reshape(D, 1))
